# Initial kernel scaffold; baseline (speedup 1.0000x reference)
import jax
import jax.numpy as jnp
from jax.experimental import pallas as pl


def kernel(x, edge_index, W1, b1, W2, b2, fcW, fcb):
    raise NotImplementedError("write your pallas kernel here")


# speedup vs baseline: 35.2580x; 35.2580x over previous
import functools

import jax
import jax.numpy as jnp
from jax import lax
from jax.experimental import pallas as pl
from jax.experimental.pallas import tpu as pltpu
from jax.experimental.pallas import tpu_sc as plsc

NN = 10000
NP = 10240
EE = 320000
FF = 16
NC = 2
NS = 16
NW = NC * NS
EPW = EE // NW
CH = 125
NCH = EPW // CH
RPT = NP // NS

_mesh = plsc.VectorSubcoreMesh(
    core_axis_name="c", subcore_axis_name="s", num_cores=NC, num_subcores=NS
)
_sc_params = pltpu.CompilerParams(use_tc_tiling_on_sc=False)


def _deg_body(dst_hbm, zeros_hbm, ones_hbm, out_hbm, dst_v, ones_v, acc_sh):
    cid = lax.axis_index("c")
    sid = lax.axis_index("s")
    wid = sid * NC + cid
    pltpu.sync_copy(dst_hbm.at[wid], dst_v)
    pltpu.sync_copy(zeros_hbm, acc_sh.at[pl.ds(sid * RPT, RPT)])
    pltpu.sync_copy(ones_hbm, ones_v)
    plsc.subcore_barrier()

    @pl.loop(0, NCH)
    def _chunk(j):
        pltpu.sync_copy(ones_v, acc_sh.at[dst_v.at[j]], add=True)

    plsc.subcore_barrier()
    pltpu.sync_copy(
        acc_sh.at[pl.ds(sid * RPT, RPT)],
        out_hbm.at[cid, pl.ds(sid * RPT, RPT)],
    )


_deg_call = pl.kernel(
    _deg_body,
    out_type=jax.ShapeDtypeStruct((NC, NP, FF), jnp.float32),
    mesh=_mesh,
    scratch_types=[
        pltpu.VMEM((NCH, CH), jnp.int32),
        pltpu.VMEM((CH, FF), jnp.float32),
        pltpu.VMEM_SHARED((NP, FF), jnp.float32),
    ],
    compiler_params=_sc_params,
)


def _agg_body(src_hbm, dst_hbm, g_hbm, zeros_hbm, out_hbm,
              src_v, dst_v, rows_v, acc_sh):
    cid = lax.axis_index("c")
    sid = lax.axis_index("s")
    wid = sid * NC + cid
    pltpu.sync_copy(src_hbm.at[wid], src_v)
    pltpu.sync_copy(dst_hbm.at[wid], dst_v)
    pltpu.sync_copy(zeros_hbm, acc_sh.at[pl.ds(sid * RPT, RPT)])
    plsc.subcore_barrier()

    @pl.loop(0, NCH)
    def _chunk(j):
        pltpu.sync_copy(g_hbm.at[src_v.at[j]], rows_v)
        pltpu.sync_copy(rows_v, acc_sh.at[dst_v.at[j]], add=True)

    plsc.subcore_barrier()
    pltpu.sync_copy(
        acc_sh.at[pl.ds(sid * RPT, RPT)],
        out_hbm.at[cid, pl.ds(sid * RPT, RPT)],
    )


_agg_call = pl.kernel(
    _agg_body,
    out_type=jax.ShapeDtypeStruct((NC, NP, FF), jnp.float32),
    mesh=_mesh,
    scratch_types=[
        pltpu.VMEM((NCH, CH), jnp.int32),
        pltpu.VMEM((NCH, CH), jnp.int32),
        pltpu.VMEM((CH, FF), jnp.float32),
        pltpu.VMEM_SHARED((NP, FF), jnp.float32),
    ],
    compiler_params=_sc_params,
)


def _tc_a_body(x_ref, w1_ref, degp_ref, dinv_ref, g1_ref):
    deg = 1.0 + degp_ref[0] + degp_ref[1]
    dinv = lax.rsqrt(deg)
    dinv_ref[...] = dinv
    h1 = jnp.dot(x_ref[...], w1_ref[...], preferred_element_type=jnp.float32)
    g1_ref[...] = h1 * dinv


_tc_a = pl.pallas_call(
    _tc_a_body,
    out_shape=[
        jax.ShapeDtypeStruct((NP, FF), jnp.float32),
        jax.ShapeDtypeStruct((NP, FF), jnp.float32),
    ],
)


def _tc_b_body(aggp_ref, g1_ref, dinv_ref, b1_ref, w2_ref, g2_ref):
    s1 = aggp_ref[0] + aggp_ref[1] + g1_ref[...]
    h = jnp.maximum(b1_ref[...] + dinv_ref[...] * s1, 0.0)
    g2 = jnp.dot(h, w2_ref[...], preferred_element_type=jnp.float32)
    g2_ref[...] = g2 * dinv_ref[...]


_tc_b = pl.pallas_call(
    _tc_b_body,
    out_shape=jax.ShapeDtypeStruct((NP, FF), jnp.float32),
)


def _tc_c_body(aggp_ref, g2_ref, dinv_ref, b2_ref, fcw_ref, fcb_ref, out_ref):
    s2 = aggp_ref[0] + aggp_ref[1] + g2_ref[...]
    h = jnp.maximum(b2_ref[...] + dinv_ref[...] * s2, 0.0)
    pooled = jnp.sum(h[:NN, :], axis=0, keepdims=True) * (1.0 / NN)
    z = jnp.sum(pooled * fcw_ref[...], axis=1, keepdims=True) + fcb_ref[...]
    out_ref[...] = jax.nn.sigmoid(z)


_tc_c = pl.pallas_call(
    _tc_c_body,
    out_shape=jax.ShapeDtypeStruct((1, 1), jnp.float32),
)


@jax.jit
def kernel(x, edge_index, W1, b1, W2, b2, fcW, fcb):
    xp = jnp.pad(x, ((0, NP - NN), (0, 0)))
    src = edge_index[0].reshape(NW, NCH, CH)
    dst = edge_index[1].reshape(NW, NCH, CH)
    zeros_stripe = jnp.zeros((RPT, FF), jnp.float32)
    ones_rows = jnp.ones((CH, FF), jnp.float32)

    degp = _deg_call(dst, zeros_stripe, ones_rows)
    dinv_rep, g1 = _tc_a(xp, W1, degp)
    agg1p = _agg_call(src, dst, g1, zeros_stripe)
    w2p = jnp.pad(W2, ((0, 0), (0, FF - W2.shape[1])))
    g2 = _tc_b(agg1p, g1, dinv_rep, b1.reshape(1, FF), w2p)
    agg2p = _agg_call(src, dst, g2, zeros_stripe)
    b2p = jnp.pad(b2, (0, FF - b2.shape[0])).reshape(1, FF)
    fcwp = jnp.pad(fcW[:, 0], (0, FF - fcW.shape[0])).reshape(1, FF)
    out = _tc_c(agg2p, g2, dinv_rep, b2p, fcwp, fcb.reshape(1, 1))
    return out.reshape(1)


# speedup vs baseline: 57.5449x; 1.6321x over previous
import functools

import jax
import jax.numpy as jnp
from jax import lax
from jax.experimental import pallas as pl
from jax.experimental.pallas import tpu as pltpu
from jax.experimental.pallas import tpu_sc as plsc

NN = 10000
NP = 10240
EE = 320000
FF = 16
NC = 2
NS = 16
NW = NC * NS
EPW = EE // NW
CH = 125
NCH = EPW // CH
RPT = NP // NS

_mesh = plsc.VectorSubcoreMesh(
    core_axis_name="c", subcore_axis_name="s", num_cores=NC, num_subcores=NS
)
_sc_params = pltpu.CompilerParams(use_tc_tiling_on_sc=False)


def _deg_body(dst_hbm, zeros_hbm, ones_hbm, out_hbm, dst_v, ones_v, acc_sh,
              ssem):
    cid = lax.axis_index("c")
    sid = lax.axis_index("s")
    wid = sid * NC + cid
    pltpu.sync_copy(dst_hbm.at[wid], dst_v)
    pltpu.sync_copy(zeros_hbm, acc_sh.at[pl.ds(sid * RPT, RPT)])
    pltpu.sync_copy(ones_hbm, ones_v)
    plsc.subcore_barrier()

    @pl.loop(0, NCH)
    def _chunk(j):
        pltpu.async_copy(ones_v, acc_sh.at[dst_v.at[j]], ssem, add=True)

    @pl.loop(0, NCH)
    def _drain(j):
        pltpu.make_async_copy(ones_v, acc_sh.at[dst_v.at[j]], ssem).wait()

    plsc.subcore_barrier()
    pltpu.sync_copy(
        acc_sh.at[pl.ds(sid * RPT, RPT)],
        out_hbm.at[cid, pl.ds(sid * RPT, RPT)],
    )


_deg_call = pl.kernel(
    _deg_body,
    out_type=jax.ShapeDtypeStruct((NC, NP, FF), jnp.float32),
    mesh=_mesh,
    scratch_types=[
        pltpu.VMEM((NCH, CH), jnp.int32),
        pltpu.VMEM((CH, FF), jnp.float32),
        pltpu.VMEM_SHARED((NP, FF), jnp.float32),
        pltpu.SemaphoreType.DMA,
    ],
    compiler_params=_sc_params,
)


KG = 4
NG = NCH // KG


def _agg_body(src_hbm, dst_hbm, g_hbm, zeros_hbm, out_hbm,
              src_v, dst_v, rows_v, acc_sh, gsem0, gsem1):
    cid = lax.axis_index("c")
    sid = lax.axis_index("s")
    wid = sid * NC + cid
    pltpu.sync_copy(src_hbm.at[wid], src_v)
    pltpu.sync_copy(dst_hbm.at[wid], dst_v)
    pltpu.sync_copy(zeros_hbm, acc_sh.at[pl.ds(sid * RPT, RPT)])
    plsc.subcore_barrier()

    def fire(g, half, sem):
        for b in range(KG):
            pltpu.async_copy(
                g_hbm.at[src_v.at[g * KG + b]], rows_v.at[half * KG + b], sem
            )

    def drain(g, half, sem):
        for b in range(KG):
            pltpu.make_async_copy(
                g_hbm.at[src_v.at[g * KG + b]], rows_v.at[half * KG + b], sem
            ).wait()

    def scat(g, half):
        for b in range(KG):
            pltpu.sync_copy(
                rows_v.at[half * KG + b], acc_sh.at[dst_v.at[g * KG + b]],
                add=True,
            )

    fire(0, 0, gsem0)

    @pl.loop(0, NG, step=2)
    def _pair(g):
        fire(g + 1, 1, gsem1)
        drain(g, 0, gsem0)
        scat(g, 0)

        @pl.when(g + 2 < NG)
        def _():
            fire(g + 2, 0, gsem0)

        drain(g + 1, 1, gsem1)
        scat(g + 1, 1)

    plsc.subcore_barrier()
    pltpu.sync_copy(
        acc_sh.at[pl.ds(sid * RPT, RPT)],
        out_hbm.at[cid, pl.ds(sid * RPT, RPT)],
    )


_agg_call = pl.kernel(
    _agg_body,
    out_type=jax.ShapeDtypeStruct((NC, NP, FF), jnp.float32),
    mesh=_mesh,
    scratch_types=[
        pltpu.VMEM((NCH, CH), jnp.int32),
        pltpu.VMEM((NCH, CH), jnp.int32),
        pltpu.VMEM((2 * KG, CH, FF), jnp.float32),
        pltpu.VMEM_SHARED((NP, FF), jnp.float32),
        pltpu.SemaphoreType.DMA,
        pltpu.SemaphoreType.DMA,
    ],
    compiler_params=_sc_params,
)


def _tc_a_body(x_ref, w1_ref, degp_ref, dinv_ref, g1_ref):
    deg = 1.0 + degp_ref[0] + degp_ref[1]
    dinv = lax.rsqrt(deg)
    dinv_ref[...] = dinv
    h1 = jnp.dot(x_ref[...], w1_ref[...], preferred_element_type=jnp.float32)
    g1_ref[...] = h1 * dinv


_tc_a = pl.pallas_call(
    _tc_a_body,
    out_shape=[
        jax.ShapeDtypeStruct((NP, FF), jnp.float32),
        jax.ShapeDtypeStruct((NP, FF), jnp.float32),
    ],
)


def _tc_b_body(aggp_ref, g1_ref, dinv_ref, b1_ref, w2_ref, g2_ref):
    s1 = aggp_ref[0] + aggp_ref[1] + g1_ref[...]
    h = jnp.maximum(b1_ref[...] + dinv_ref[...] * s1, 0.0)
    g2 = jnp.dot(h, w2_ref[...], preferred_element_type=jnp.float32)
    g2_ref[...] = g2 * dinv_ref[...]


_tc_b = pl.pallas_call(
    _tc_b_body,
    out_shape=jax.ShapeDtypeStruct((NP, FF), jnp.float32),
)


def _tc_c_body(aggp_ref, g2_ref, dinv_ref, b2_ref, fcw_ref, fcb_ref, out_ref):
    s2 = aggp_ref[0] + aggp_ref[1] + g2_ref[...]
    h = jnp.maximum(b2_ref[...] + dinv_ref[...] * s2, 0.0)
    pooled = jnp.sum(h[:NN, :], axis=0, keepdims=True) * (1.0 / NN)
    z = jnp.sum(pooled * fcw_ref[...], axis=1, keepdims=True) + fcb_ref[...]
    out_ref[...] = jax.nn.sigmoid(z)


_tc_c = pl.pallas_call(
    _tc_c_body,
    out_shape=jax.ShapeDtypeStruct((1, 1), jnp.float32),
)


@jax.jit
def kernel(x, edge_index, W1, b1, W2, b2, fcW, fcb):
    xp = jnp.pad(x, ((0, NP - NN), (0, 0)))
    src = edge_index[0].reshape(NW, NCH, CH)
    dst = edge_index[1].reshape(NW, NCH, CH)
    zeros_stripe = jnp.zeros((RPT, FF), jnp.float32)
    ones_rows = jnp.ones((CH, FF), jnp.float32)

    degp = _deg_call(dst, zeros_stripe, ones_rows)
    dinv_rep, g1 = _tc_a(xp, W1, degp)
    agg1p = _agg_call(src, dst, g1, zeros_stripe)
    w2p = jnp.pad(W2, ((0, 0), (0, FF - W2.shape[1])))
    g2 = _tc_b(agg1p, g1, dinv_rep, b1.reshape(1, FF), w2p)
    agg2p = _agg_call(src, dst, g2, zeros_stripe)
    b2p = jnp.pad(b2, (0, FF - b2.shape[0])).reshape(1, FF)
    fcwp = jnp.pad(fcW[:, 0], (0, FF - fcW.shape[0])).reshape(1, FF)
    out = _tc_c(agg2p, g2, dinv_rep, b2p, fcwp, fcb.reshape(1, 1))
    return out.reshape(1)


# speedup vs baseline: 57.7530x; 1.0036x over previous
import functools

import jax
import jax.numpy as jnp
from jax import lax
from jax.experimental import pallas as pl
from jax.experimental.pallas import tpu as pltpu
from jax.experimental.pallas import tpu_sc as plsc

NN = 10000
NP = 10240
EE = 320000
FF = 16
NC = 2
NS = 16
NW = NC * NS
EPW = EE // NW
CH = 125
NCH = EPW // CH
RPT = NP // NS

_mesh = plsc.VectorSubcoreMesh(
    core_axis_name="c", subcore_axis_name="s", num_cores=NC, num_subcores=NS
)
_sc_params = pltpu.CompilerParams(use_tc_tiling_on_sc=False)


NR = NP // 16
NID = NR // 128


def _deg_body(dst_hbm, iden_hbm, zeros_hbm, out_hbm, dst_v, iden_v, loc_v,
              acc_sh, ssem):
    cid = lax.axis_index("c")
    sid = lax.axis_index("s")
    wid = sid * NC + cid
    pltpu.sync_copy(dst_hbm.at[wid], dst_v)
    pltpu.sync_copy(iden_hbm, iden_v)
    pltpu.sync_copy(zeros_hbm, acc_sh.at[pl.ds(sid * (NR // NS), NR // NS)])

    @pl.loop(0, NR)
    def _zero(i):
        loc_v[i, :] = jnp.zeros((16,), jnp.float32)

    ones16 = jnp.ones((16,), jnp.float32)

    @pl.loop(0, EPW // 16)
    def _count(i):
        dv = dst_v[pl.ds(i * 16, 16)]
        plsc.addupdate_scatter(
            loc_v, [lax.shift_right_logical(dv, 4), lax.bitwise_and(dv, 15)],
            ones16,
        )

    plsc.subcore_barrier()

    for j in range(NID):
        pltpu.async_copy(
            loc_v.at[pl.ds(j * 128, 128)], acc_sh.at[iden_v.at[j]], ssem,
            add=True,
        )
    for j in range(NID):
        pltpu.make_async_copy(
            loc_v.at[pl.ds(j * 128, 128)], acc_sh.at[iden_v.at[j]], ssem
        ).wait()

    plsc.subcore_barrier()
    pltpu.sync_copy(
        acc_sh.at[pl.ds(sid * (NR // NS), NR // NS)],
        out_hbm.at[cid, pl.ds(sid * (NR // NS), NR // NS)],
    )


_deg_call = pl.kernel(
    _deg_body,
    out_type=jax.ShapeDtypeStruct((NC, NR, 16), jnp.float32),
    mesh=_mesh,
    scratch_types=[
        pltpu.VMEM((EPW,), jnp.int32),
        pltpu.VMEM((NID, 128), jnp.int32),
        pltpu.VMEM((NR, 16), jnp.float32),
        pltpu.VMEM_SHARED((NR, 16), jnp.float32),
        pltpu.SemaphoreType.DMA,
    ],
    compiler_params=pltpu.CompilerParams(
        use_tc_tiling_on_sc=False, needs_layout_passes=False
    ),
)


KG = 4
NG = NCH // KG


def _agg_body(src_hbm, dst_hbm, g_hbm, zeros_hbm, out_hbm,
              src_v, dst_v, rows_v, acc_sh, gsem0, gsem1):
    cid = lax.axis_index("c")
    sid = lax.axis_index("s")
    wid = sid * NC + cid
    pltpu.sync_copy(src_hbm.at[wid], src_v)
    pltpu.sync_copy(dst_hbm.at[wid], dst_v)
    pltpu.sync_copy(zeros_hbm, acc_sh.at[pl.ds(sid * RPT, RPT)])
    plsc.subcore_barrier()

    def fire(g, half, sem):
        for b in range(KG):
            pltpu.async_copy(
                g_hbm.at[src_v.at[g * KG + b]], rows_v.at[half * KG + b], sem
            )

    def drain(g, half, sem):
        for b in range(KG):
            pltpu.make_async_copy(
                g_hbm.at[src_v.at[g * KG + b]], rows_v.at[half * KG + b], sem
            ).wait()

    def scat(g, half):
        for b in range(KG):
            pltpu.sync_copy(
                rows_v.at[half * KG + b], acc_sh.at[dst_v.at[g * KG + b]],
                add=True,
            )

    fire(0, 0, gsem0)

    @pl.loop(0, NG, step=2)
    def _pair(g):
        fire(g + 1, 1, gsem1)
        drain(g, 0, gsem0)
        scat(g, 0)

        @pl.when(g + 2 < NG)
        def _():
            fire(g + 2, 0, gsem0)

        drain(g + 1, 1, gsem1)
        scat(g + 1, 1)

    plsc.subcore_barrier()
    pltpu.sync_copy(
        acc_sh.at[pl.ds(sid * RPT, RPT)],
        out_hbm.at[cid, pl.ds(sid * RPT, RPT)],
    )


_agg_call = pl.kernel(
    _agg_body,
    out_type=jax.ShapeDtypeStruct((NC, NP, FF), jnp.float32),
    mesh=_mesh,
    scratch_types=[
        pltpu.VMEM((NCH, CH), jnp.int32),
        pltpu.VMEM((NCH, CH), jnp.int32),
        pltpu.VMEM((2 * KG, CH, FF), jnp.float32),
        pltpu.VMEM_SHARED((NP, FF), jnp.float32),
        pltpu.SemaphoreType.DMA,
        pltpu.SemaphoreType.DMA,
    ],
    compiler_params=_sc_params,
)


def _tc_a_body(x_ref, w1_ref, degp_ref, dinv_ref, g1_ref):
    deg = 1.0 + degp_ref[0] + degp_ref[1]
    dinv = lax.rsqrt(deg)
    dinv_ref[...] = dinv
    h1 = jnp.dot(x_ref[...], w1_ref[...], preferred_element_type=jnp.float32)
    g1_ref[...] = h1 * dinv


_tc_a = pl.pallas_call(
    _tc_a_body,
    out_shape=[
        jax.ShapeDtypeStruct((NP, 1), jnp.float32),
        jax.ShapeDtypeStruct((NP, FF), jnp.float32),
    ],
)


def _tc_b_body(aggp_ref, g1_ref, dinv_ref, b1_ref, w2_ref, g2_ref):
    s1 = aggp_ref[0] + aggp_ref[1] + g1_ref[...]
    h = jnp.maximum(b1_ref[...] + dinv_ref[...] * s1, 0.0)
    g2 = jnp.dot(h, w2_ref[...], preferred_element_type=jnp.float32)
    g2_ref[...] = g2 * dinv_ref[...]


_tc_b = pl.pallas_call(
    _tc_b_body,
    out_shape=jax.ShapeDtypeStruct((NP, FF), jnp.float32),
)


def _tc_c_body(aggp_ref, g2_ref, dinv_ref, b2_ref, fcw_ref, fcb_ref, out_ref):
    s2 = aggp_ref[0] + aggp_ref[1] + g2_ref[...]
    h = jnp.maximum(b2_ref[...] + dinv_ref[...] * s2, 0.0)
    pooled = jnp.sum(h[:NN, :], axis=0, keepdims=True) * (1.0 / NN)
    z = jnp.sum(pooled * fcw_ref[...], axis=1, keepdims=True) + fcb_ref[...]
    out_ref[...] = jax.nn.sigmoid(z)


_tc_c = pl.pallas_call(
    _tc_c_body,
    out_shape=jax.ShapeDtypeStruct((1, 1), jnp.float32),
)


@jax.jit
def kernel(x, edge_index, W1, b1, W2, b2, fcW, fcb):
    xp = jnp.pad(x, ((0, NP - NN), (0, 0)))
    src = edge_index[0].reshape(NW, NCH, CH)
    dst = edge_index[1].reshape(NW, NCH, CH)
    zeros_stripe = jnp.zeros((RPT, FF), jnp.float32)

    iden = jnp.arange(NR, dtype=jnp.int32).reshape(NID, 128)
    zeros_deg = jnp.zeros((NR // NS, 16), jnp.float32)
    degp = _deg_call(dst.reshape(NW, EPW), iden, zeros_deg)
    dinv_col, g1 = _tc_a(xp, W1, degp.reshape(NC, NP, 1))
    agg1p = _agg_call(src, dst, g1, zeros_stripe)
    w2p = jnp.pad(W2, ((0, 0), (0, FF - W2.shape[1])))
    g2 = _tc_b(agg1p, g1, dinv_col, b1.reshape(1, FF), w2p)
    agg2p = _agg_call(src, dst, g2, zeros_stripe)
    b2p = jnp.pad(b2, (0, FF - b2.shape[0])).reshape(1, FF)
    fcwp = jnp.pad(fcW[:, 0], (0, FF - fcW.shape[0])).reshape(1, FF)
    out = _tc_c(agg2p, g2, dinv_col, b2p, fcwp, fcb.reshape(1, 1))
    return out.reshape(1)


# speedup vs baseline: 59.1617x; 1.0244x over previous
import functools

import jax
import jax.numpy as jnp
from jax import lax
from jax.experimental import pallas as pl
from jax.experimental.pallas import tpu as pltpu
from jax.experimental.pallas import tpu_sc as plsc

NN = 10000
NP = 10240
EE = 320000
FF = 16
NC = 2
NS = 16
NW = NC * NS
EPW = EE // NW
CH = 125
NCH = EPW // CH
RPT = NP // NS

_mesh = plsc.VectorSubcoreMesh(
    core_axis_name="c", subcore_axis_name="s", num_cores=NC, num_subcores=NS
)
_sc_params = pltpu.CompilerParams(use_tc_tiling_on_sc=False)


NR = NP // 16
NID = NR // 128


def _deg_body(edge_hbm, iden_hbm, zeros_hbm, out_hbm, dst_v, iden_v, loc_v,
              acc_sh, ssem):
    cid = lax.axis_index("c")
    sid = lax.axis_index("s")
    wid = sid * NC + cid
    pltpu.sync_copy(edge_hbm.at[1, wid], dst_v)
    pltpu.sync_copy(iden_hbm, iden_v)
    pltpu.sync_copy(zeros_hbm, acc_sh.at[pl.ds(sid * (NR // NS), NR // NS)])

    @pl.loop(0, NR)
    def _zero(i):
        loc_v[i, :] = jnp.zeros((16,), jnp.float32)

    ones16 = jnp.ones((16,), jnp.float32)
    lanes = lax.iota(jnp.int32, 16)
    tail_mask = lanes < (CH - 112)

    @pl.loop(0, NCH)
    def _count(j):
        for c in range(0, 112, 16):
            dv = dst_v[j, pl.ds(c, 16)]
            plsc.addupdate_scatter(
                loc_v,
                [lax.shift_right_logical(dv, 4), lax.bitwise_and(dv, 15)],
                ones16,
            )
        dvt = plsc.load_gather(dst_v, [jnp.full((16,), j, jnp.int32),
                                       jnp.minimum(112 + lanes, CH - 1)])
        plsc.addupdate_scatter(
            loc_v,
            [lax.shift_right_logical(dvt, 4), lax.bitwise_and(dvt, 15)],
            ones16, mask=tail_mask,
        )

    plsc.subcore_barrier()

    for j in range(NID):
        pltpu.async_copy(
            loc_v.at[pl.ds(j * 128, 128)], acc_sh.at[iden_v.at[j]], ssem,
            add=True,
        )
    for j in range(NID):
        pltpu.make_async_copy(
            loc_v.at[pl.ds(j * 128, 128)], acc_sh.at[iden_v.at[j]], ssem
        ).wait()

    plsc.subcore_barrier()
    pltpu.sync_copy(
        acc_sh.at[pl.ds(sid * (NR // NS), NR // NS)],
        out_hbm.at[cid, pl.ds(sid * (NR // NS), NR // NS)],
    )


_deg_call = pl.kernel(
    _deg_body,
    out_type=jax.ShapeDtypeStruct((NC, NR, 16), jnp.float32),
    mesh=_mesh,
    scratch_types=[
        pltpu.VMEM((NCH, CH), jnp.int32),
        pltpu.VMEM((NID, 128), jnp.int32),
        pltpu.VMEM((NR, 16), jnp.float32),
        pltpu.VMEM_SHARED((NR, 16), jnp.float32),
        pltpu.SemaphoreType.DMA,
    ],
    compiler_params=pltpu.CompilerParams(
        use_tc_tiling_on_sc=False, needs_layout_passes=False
    ),
)


KG = 4
NG = NCH // KG


def _agg_body(edge_hbm, g_hbm, zeros_hbm, out_hbm,
              src_v, dst_v, rows_v, acc_sh, gsem0, gsem1):
    cid = lax.axis_index("c")
    sid = lax.axis_index("s")
    wid = sid * NC + cid
    pltpu.sync_copy(edge_hbm.at[0, wid], src_v)
    pltpu.sync_copy(edge_hbm.at[1, wid], dst_v)
    pltpu.sync_copy(zeros_hbm, acc_sh.at[pl.ds(sid * RPT, RPT)])
    plsc.subcore_barrier()

    def fire(g, half, sem):
        for b in range(KG):
            pltpu.async_copy(
                g_hbm.at[src_v.at[g * KG + b]], rows_v.at[half * KG + b], sem
            )

    def drain(g, half, sem):
        for b in range(KG):
            pltpu.make_async_copy(
                g_hbm.at[src_v.at[g * KG + b]], rows_v.at[half * KG + b], sem
            ).wait()

    def scat(g, half):
        for b in range(KG):
            pltpu.sync_copy(
                rows_v.at[half * KG + b], acc_sh.at[dst_v.at[g * KG + b]],
                add=True,
            )

    fire(0, 0, gsem0)

    @pl.loop(0, NG, step=2)
    def _pair(g):
        fire(g + 1, 1, gsem1)
        drain(g, 0, gsem0)
        scat(g, 0)

        @pl.when(g + 2 < NG)
        def _():
            fire(g + 2, 0, gsem0)

        drain(g + 1, 1, gsem1)
        scat(g + 1, 1)

    plsc.subcore_barrier()
    pltpu.sync_copy(
        acc_sh.at[pl.ds(sid * RPT, RPT)],
        out_hbm.at[cid, pl.ds(sid * RPT, RPT)],
    )


_agg_call = pl.kernel(
    _agg_body,
    out_type=jax.ShapeDtypeStruct((NC, NP, FF), jnp.float32),
    mesh=_mesh,
    scratch_types=[
        pltpu.VMEM((NCH, CH), jnp.int32),
        pltpu.VMEM((NCH, CH), jnp.int32),
        pltpu.VMEM((2 * KG, CH, FF), jnp.float32),
        pltpu.VMEM_SHARED((NP, FF), jnp.float32),
        pltpu.SemaphoreType.DMA,
        pltpu.SemaphoreType.DMA,
    ],
    compiler_params=_sc_params,
)


def _tc_a_body(x_ref, w1_ref, degp_ref, dinv_ref, g1_ref):
    deg = 1.0 + degp_ref[0] + degp_ref[1]
    dinv = lax.rsqrt(deg)
    dinv_ref[...] = dinv
    h1 = jnp.dot(x_ref[...], w1_ref[...], preferred_element_type=jnp.float32)
    g1_ref[...] = h1 * dinv


_tc_a = pl.pallas_call(
    _tc_a_body,
    out_shape=[
        jax.ShapeDtypeStruct((NP, 1), jnp.float32),
        jax.ShapeDtypeStruct((NP, FF), jnp.float32),
    ],
)


def _tc_b_body(aggp_ref, g1_ref, dinv_ref, b1_ref, w2_ref, g2_ref):
    s1 = aggp_ref[0] + aggp_ref[1] + g1_ref[...]
    h = jnp.maximum(b1_ref[...] + dinv_ref[...] * s1, 0.0)
    g2 = jnp.dot(h, w2_ref[...], preferred_element_type=jnp.float32)
    g2_ref[...] = g2 * dinv_ref[...]


_tc_b = pl.pallas_call(
    _tc_b_body,
    out_shape=jax.ShapeDtypeStruct((NP, FF), jnp.float32),
)


def _tc_c_body(aggp_ref, g2_ref, dinv_ref, b2_ref, fcw_ref, fcb_ref, out_ref):
    s2 = aggp_ref[0] + aggp_ref[1] + g2_ref[...]
    h = jnp.maximum(b2_ref[...] + dinv_ref[...] * s2, 0.0)
    pooled = jnp.sum(h[:NN, :], axis=0, keepdims=True) * (1.0 / NN)
    z = jnp.sum(pooled * fcw_ref[...], axis=1, keepdims=True) + fcb_ref[...]
    out_ref[...] = jax.nn.sigmoid(z)


_tc_c = pl.pallas_call(
    _tc_c_body,
    out_shape=jax.ShapeDtypeStruct((1, 1), jnp.float32),
)


@jax.jit
def kernel(x, edge_index, W1, b1, W2, b2, fcW, fcb):
    xp = jnp.pad(x, ((0, NP - NN), (0, 0)))
    edge4 = edge_index.reshape(2, NW, NCH, CH)
    zeros_stripe = jnp.zeros((RPT, FF), jnp.float32)

    iden = jnp.arange(NR, dtype=jnp.int32).reshape(NID, 128)
    zeros_deg = jnp.zeros((NR // NS, 16), jnp.float32)
    degp = _deg_call(edge4, iden, zeros_deg)
    dinv_col, g1 = _tc_a(xp, W1, degp.reshape(NC, NP, 1))
    agg1p = _agg_call(edge4, g1, zeros_stripe)
    w2p = jnp.pad(W2, ((0, 0), (0, FF - W2.shape[1])))
    g2 = _tc_b(agg1p, g1, dinv_col, b1.reshape(1, FF), w2p)
    agg2p = _agg_call(edge4, g2, zeros_stripe)
    b2p = jnp.pad(b2, (0, FF - b2.shape[0])).reshape(1, FF)
    fcwp = jnp.pad(fcW[:, 0], (0, FF - fcW.shape[0])).reshape(1, FF)
    out = _tc_c(agg2p, g2, dinv_col, b2p, fcwp, fcb.reshape(1, 1))
    return out.reshape(1)


# speedup vs baseline: 67.7834x; 1.1457x over previous
import jax
import jax.numpy as jnp
from jax import lax
from jax.experimental import pallas as pl
from jax.experimental.pallas import tpu as pltpu
from jax.experimental.pallas import tpu_sc as plsc

NN = 10000
NP = 10240
EE = 320000
FF = 16
NC = 2
NS = 16
NW = NC * NS
EPW = EE // NW
CH = 125
NCH = EPW // CH
RPT = NP // NS
NR = NP // 16
SPR = NR // NS
NID = NR // 128

_mesh = plsc.VectorSubcoreMesh(
    core_axis_name="c", subcore_axis_name="s", num_cores=NC, num_subcores=NS
)
_sc_params = pltpu.CompilerParams(
    use_tc_tiling_on_sc=False, needs_layout_passes=False
)
_sc_params_nl = _sc_params

KG = 4
NG = NCH // KG


def _rsqrt_nr(d):
    i = plsc.bitcast(d, jnp.int32)
    y = plsc.bitcast(
        jnp.full((16,), 0x5F3759DF, jnp.int32) - lax.shift_right_logical(i, 1),
        jnp.float32,
    )
    for _ in range(3):
        y = y * (1.5 - 0.5 * d * y * y)
    return y


def _dinv_packed(dp0_v, dp1_v, dpk_v):
    @pl.loop(0, SPR)
    def _(r):
        dpk_v[r, :] = _rsqrt_nr(1.0 + dp0_v[r] + dp1_v[r])


def _edge_pipeline(gsrc, src_v, dst_v, rows_v, acc_sh, gsem0, gsem1):
    def fire(g, half, sem):
        for b in range(KG):
            pltpu.async_copy(
                gsrc.at[src_v.at[g * KG + b]], rows_v.at[half * KG + b], sem
            )

    def drain(g, half, sem):
        for b in range(KG):
            pltpu.make_async_copy(
                gsrc.at[src_v.at[g * KG + b]], rows_v.at[half * KG + b], sem
            ).wait()

    def scat(g, half):
        for b in range(KG):
            pltpu.sync_copy(
                rows_v.at[half * KG + b], acc_sh.at[dst_v.at[g * KG + b]],
                add=True,
            )

    fire(0, 0, gsem0)

    @pl.loop(0, NG, step=2)
    def _pair(g):
        fire(g + 1, 1, gsem1)
        drain(g, 0, gsem0)
        scat(g, 0)

        @pl.when(g + 2 < NG)
        def _():
            fire(g + 2, 0, gsem0)

        drain(g + 1, 1, gsem1)
        scat(g + 1, 1)


def _mm1_body(x_ref, w1_ref, h1_ref):
    h1_ref[pl.ds(0, NN), :] = jnp.dot(
        x_ref[...], w1_ref[...], preferred_element_type=jnp.float32
    )
    h1_ref[pl.ds(NN, NP - NN), :] = jnp.zeros((NP - NN, FF), jnp.float32)


_mm1 = pl.pallas_call(
    _mm1_body, out_shape=jax.ShapeDtypeStruct((NP, FF), jnp.float32)
)


def _deg_body(edge_hbm, iden_hbm, zeros_hbm, out_hbm, dst_v, iden_v, loc_v,
              acc_sh, ssem):
    cid = lax.axis_index("c")
    sid = lax.axis_index("s")
    wid = sid * NC + cid
    pltpu.sync_copy(edge_hbm.at[1, wid], dst_v)
    pltpu.sync_copy(iden_hbm, iden_v)
    pltpu.sync_copy(zeros_hbm, acc_sh.at[pl.ds(sid * SPR, SPR)])

    @pl.loop(0, NR)
    def _zero(i):
        loc_v[i, :] = jnp.zeros((16,), jnp.float32)

    ones16 = jnp.ones((16,), jnp.float32)
    lanes = lax.iota(jnp.int32, 16)
    tail_mask = lanes < (CH - 112)

    @pl.loop(0, NCH)
    def _count(j):
        for c in range(0, 112, 16):
            dv = dst_v[j, pl.ds(c, 16)]
            plsc.addupdate_scatter(
                loc_v,
                [lax.shift_right_logical(dv, 4), lax.bitwise_and(dv, 15)],
                ones16,
            )
        dvt = plsc.load_gather(dst_v, [jnp.full((16,), j, jnp.int32),
                                       jnp.minimum(112 + lanes, CH - 1)])
        plsc.addupdate_scatter(
            loc_v,
            [lax.shift_right_logical(dvt, 4), lax.bitwise_and(dvt, 15)],
            ones16, mask=tail_mask,
        )

    plsc.subcore_barrier()

    for j in range(NID):
        pltpu.async_copy(
            loc_v.at[pl.ds(j * 128, 128)], acc_sh.at[iden_v.at[j]], ssem,
            add=True,
        )
    for j in range(NID):
        pltpu.make_async_copy(
            loc_v.at[pl.ds(j * 128, 128)], acc_sh.at[iden_v.at[j]], ssem
        ).wait()

    plsc.subcore_barrier()
    pltpu.sync_copy(
        acc_sh.at[pl.ds(sid * SPR, SPR)],
        out_hbm.at[cid, pl.ds(sid * SPR, SPR)],
    )


_deg_call = pl.kernel(
    _deg_body,
    out_type=jax.ShapeDtypeStruct((NC, NR, 16), jnp.float32),
    mesh=_mesh,
    scratch_types=[
        pltpu.VMEM((NCH, CH), jnp.int32),
        pltpu.VMEM((NID, 128), jnp.int32),
        pltpu.VMEM((NR, 16), jnp.float32),
        pltpu.VMEM_SHARED((NR, 16), jnp.float32),
        pltpu.SemaphoreType.DMA,
    ],
    compiler_params=_sc_params_nl,
)


def _agg1_body(edge_hbm, h1_hbm, degp_hbm, zeros_hbm, aggp_hbm, g1_hbm,
               src_v, dst_v, rows_v, dp0_v, dp1_v, dpk_v, g1_v, acc_sh, g_sh,
               gsem0, gsem1):
    cid = lax.axis_index("c")
    sid = lax.axis_index("s")
    wid = sid * NC + cid
    pltpu.sync_copy(edge_hbm.at[0, wid], src_v)
    pltpu.sync_copy(edge_hbm.at[1, wid], dst_v)
    pltpu.sync_copy(zeros_hbm, acc_sh.at[pl.ds(sid * RPT, RPT)])
    pltpu.sync_copy(degp_hbm.at[0, pl.ds(sid * SPR, SPR)], dp0_v)
    pltpu.sync_copy(degp_hbm.at[1, pl.ds(sid * SPR, SPR)], dp1_v)
    pltpu.sync_copy(h1_hbm.at[pl.ds(sid * RPT, RPT)], g1_v)
    _dinv_packed(dp0_v, dp1_v, dpk_v)

    @pl.loop(0, SPR)
    def _scale(r):
        drow = dpk_v[r, :]
        for c in range(16):
            n = r * 16 + c
            g1_v[n, :] = g1_v[n, :] * drow[c]

    pltpu.sync_copy(g1_v, g_sh.at[pl.ds(sid * RPT, RPT)])
    pltpu.sync_copy(g1_v, g1_hbm.at[pl.ds(sid * RPT, RPT)])
    plsc.subcore_barrier()

    _edge_pipeline(g_sh, src_v, dst_v, rows_v, acc_sh, gsem0, gsem1)

    plsc.subcore_barrier()
    pltpu.sync_copy(
        acc_sh.at[pl.ds(sid * RPT, RPT)],
        aggp_hbm.at[cid, pl.ds(sid * RPT, RPT)],
    )


_agg1_call = pl.kernel(
    _agg1_body,
    out_type=[
        jax.ShapeDtypeStruct((NC, NP, FF), jnp.float32),
        jax.ShapeDtypeStruct((NP, FF), jnp.float32),
    ],
    mesh=_mesh,
    scratch_types=[
        pltpu.VMEM((NCH, CH), jnp.int32),
        pltpu.VMEM((NCH, CH), jnp.int32),
        pltpu.VMEM((2 * KG, CH, FF), jnp.float32),
        pltpu.VMEM((SPR, 16), jnp.float32),
        pltpu.VMEM((SPR, 16), jnp.float32),
        pltpu.VMEM((SPR, 16), jnp.float32),
        pltpu.VMEM((RPT, FF), jnp.float32),
        pltpu.VMEM_SHARED((NP, FF), jnp.float32),
        pltpu.VMEM_SHARED((NP, FF), jnp.float32),
        pltpu.SemaphoreType.DMA,
        pltpu.SemaphoreType.DMA,
    ],
    compiler_params=_sc_params,
)


def _agg2_body(edge_hbm, degp_hbm, aggp1_hbm, g1_hbm, b1_hbm, w2_hbm,
               zeros_hbm, aggp_hbm, g2_hbm,
               src_v, dst_v, rows_v, dp0_v, dp1_v, dpk_v, p0_v, p1_v, g_v,
               o1t_v, h2t_v, b1_v, w2_v, acc_sh, g_sh, gsem0, gsem1):
    cid = lax.axis_index("c")
    sid = lax.axis_index("s")
    wid = sid * NC + cid
    pltpu.sync_copy(edge_hbm.at[0, wid], src_v)
    pltpu.sync_copy(edge_hbm.at[1, wid], dst_v)
    pltpu.sync_copy(zeros_hbm, acc_sh.at[pl.ds(sid * RPT, RPT)])
    pltpu.sync_copy(degp_hbm.at[0, pl.ds(sid * SPR, SPR)], dp0_v)
    pltpu.sync_copy(degp_hbm.at[1, pl.ds(sid * SPR, SPR)], dp1_v)
    pltpu.sync_copy(aggp1_hbm.at[0, pl.ds(sid * RPT, RPT)], p0_v)
    pltpu.sync_copy(aggp1_hbm.at[1, pl.ds(sid * RPT, RPT)], p1_v)
    pltpu.sync_copy(g1_hbm.at[pl.ds(sid * RPT, RPT)], g_v)
    pltpu.sync_copy(b1_hbm, b1_v)
    pltpu.sync_copy(w2_hbm, w2_v)
    _dinv_packed(dp0_v, dp1_v, dpk_v)

    lanes = lax.iota(jnp.int32, 16)
    b1vec = b1_v[...]

    @pl.loop(0, SPR)
    def _comb(r):
        drow = dpk_v[r, :]
        for c in range(16):
            n = r * 16 + c
            row = jnp.maximum(
                b1vec + drow[c] * (p0_v[n, :] + p1_v[n, :] + g_v[n, :]), 0.0
            )
            plsc.store_scatter(
                o1t_v, [lanes, jnp.full((16,), n, jnp.int32)], row
            )

    for j in range(8):
        wj = w2_v[j, :]

        @pl.loop(0, RPT // 16)
        def _mm(i, _j=j, _wj=wj):
            acc = jnp.zeros((16,), jnp.float32)
            for k in range(FF):
                acc = acc + _wj[k] * o1t_v[k, pl.ds(i * 16, 16)]
            h2t_v[_j, pl.ds(i * 16, 16)] = acc

    mask8 = lanes < 8
    idx8 = jnp.minimum(lanes, 7)

    @pl.loop(0, SPR)
    def _gather_back(r):
        drow = dpk_v[r, :]
        for c in range(16):
            n = r * 16 + c
            v = plsc.load_gather(h2t_v, [idx8, jnp.full((16,), n, jnp.int32)])
            g_v[n, :] = jnp.where(mask8, v, 0.0) * drow[c]

    pltpu.sync_copy(g_v, g_sh.at[pl.ds(sid * RPT, RPT)])
    pltpu.sync_copy(g_v, g2_hbm.at[pl.ds(sid * RPT, RPT)])
    plsc.subcore_barrier()

    _edge_pipeline(g_sh, src_v, dst_v, rows_v, acc_sh, gsem0, gsem1)

    plsc.subcore_barrier()
    pltpu.sync_copy(
        acc_sh.at[pl.ds(sid * RPT, RPT)],
        aggp_hbm.at[cid, pl.ds(sid * RPT, RPT)],
    )


_agg2_call = pl.kernel(
    _agg2_body,
    out_type=[
        jax.ShapeDtypeStruct((NC, NP, FF), jnp.float32),
        jax.ShapeDtypeStruct((NP, FF), jnp.float32),
    ],
    mesh=_mesh,
    scratch_types=[
        pltpu.VMEM((NCH, CH), jnp.int32),
        pltpu.VMEM((NCH, CH), jnp.int32),
        pltpu.VMEM((2 * KG, CH, FF), jnp.float32),
        pltpu.VMEM((SPR, 16), jnp.float32),
        pltpu.VMEM((SPR, 16), jnp.float32),
        pltpu.VMEM((SPR, 16), jnp.float32),
        pltpu.VMEM((RPT, FF), jnp.float32),
        pltpu.VMEM((RPT, FF), jnp.float32),
        pltpu.VMEM((RPT, FF), jnp.float32),
        pltpu.VMEM((16, RPT), jnp.float32),
        pltpu.VMEM((8, RPT), jnp.float32),
        pltpu.VMEM((16,), jnp.float32),
        pltpu.VMEM((8, 16), jnp.float32),
        pltpu.VMEM_SHARED((NP, FF), jnp.float32),
        pltpu.VMEM_SHARED((NP, FF), jnp.float32),
        pltpu.SemaphoreType.DMA,
        pltpu.SemaphoreType.DMA,
    ],
    compiler_params=_sc_params,
)


def _fin_body(degp_hbm, aggp2_hbm, g2_hbm, b2_hbm, fcv_hbm, out_hbm,
              dp0_v, dp1_v, dpk_v, p0_v, p1_v, g_v, b2_v, fcv_v,
              tot_v, pool_sh):
    cid = lax.axis_index("c")
    sid = lax.axis_index("s")
    pltpu.sync_copy(degp_hbm.at[0, pl.ds(sid * SPR, SPR)], dp0_v)
    pltpu.sync_copy(degp_hbm.at[1, pl.ds(sid * SPR, SPR)], dp1_v)
    pltpu.sync_copy(aggp2_hbm.at[0, pl.ds(sid * RPT, RPT)], p0_v)
    pltpu.sync_copy(aggp2_hbm.at[1, pl.ds(sid * RPT, RPT)], p1_v)
    pltpu.sync_copy(g2_hbm.at[pl.ds(sid * RPT, RPT)], g_v)
    pltpu.sync_copy(b2_hbm, b2_v)
    pltpu.sync_copy(fcv_hbm, fcv_v)
    _dinv_packed(dp0_v, dp1_v, dpk_v)

    lanes = lax.iota(jnp.int32, 16)
    mask8 = lanes < 8
    b2vec = jnp.where(mask8, plsc.load_gather(b2_v, [jnp.minimum(lanes, 7)]),
                      0.0)
    nrows = jnp.where(sid == NS - 1, (NN - (NS - 1) * RPT) // 16, SPR)

    def _body(r, acc):
        drow = dpk_v[r, :]
        for c in range(16):
            n = r * 16 + c
            row = jnp.maximum(
                b2vec + drow[c] * (p0_v[n, :] + p1_v[n, :] + g_v[n, :]), 0.0
            )
            acc = acc + row
        return acc

    pooled = lax.fori_loop(0, nrows, _body, jnp.zeros((16,), jnp.float32))
    tot_v[...] = pooled
    pltpu.sync_copy(tot_v, pool_sh.at[sid])
    plsc.subcore_barrier()

    @pl.when(sid == 0)
    def _():
        pltpu.sync_copy(pool_sh, g_v.at[pl.ds(0, NS)])
        total = jnp.zeros((16,), jnp.float32)
        for r in range(NS):
            total = total + g_v[r, :]
        total = total * (1.0 / NN)
        fcv = fcv_v[...]
        z = lax.reduce_sum_p.bind(
            total * jnp.where(mask8, fcv, 0.0), axes=(0,)
        ) + fcv[8]
        zv = jnp.zeros((16,), jnp.float32) + z
        sig = 1.0 / (1.0 + jnp.exp(-zv))
        tot_v[...] = sig
        pltpu.sync_copy(tot_v, out_hbm.at[cid])


_fin_call = pl.kernel(
    _fin_body,
    out_type=jax.ShapeDtypeStruct((NC, 16), jnp.float32),
    mesh=_mesh,
    scratch_types=[
        pltpu.VMEM((SPR, 16), jnp.float32),
        pltpu.VMEM((SPR, 16), jnp.float32),
        pltpu.VMEM((SPR, 16), jnp.float32),
        pltpu.VMEM((RPT, FF), jnp.float32),
        pltpu.VMEM((RPT, FF), jnp.float32),
        pltpu.VMEM((RPT, FF), jnp.float32),
        pltpu.VMEM((8,), jnp.float32),
        pltpu.VMEM((16,), jnp.float32),
        pltpu.VMEM((16,), jnp.float32),
        pltpu.VMEM_SHARED((NS, 16), jnp.float32),
    ],
    compiler_params=_sc_params,
)


@jax.jit
def kernel(x, edge_index, W1, b1, W2, b2, fcW, fcb):
    edge4 = edge_index.reshape(2, NW, NCH, CH)
    zeros_stripe = jnp.zeros((RPT, FF), jnp.float32)
    zeros_deg = jnp.zeros((SPR, 16), jnp.float32)
    iden = jnp.arange(NR, dtype=jnp.int32).reshape(NID, 128)

    degp = _deg_call(edge4, iden, zeros_deg)
    h1 = _mm1(x, W1)
    agg1p, g1 = _agg1_call(edge4, h1, degp, zeros_stripe)
    agg2p, g2 = _agg2_call(edge4, degp, agg1p, g1, b1, W2.T, zeros_stripe)
    fcv = jnp.concatenate([fcW[:, 0], fcb, jnp.zeros((7,), jnp.float32)])
    sig = _fin_call(degp, agg2p, g2, b2, fcv)
    return sig[0, :1]


# speedup vs baseline: 75.8748x; 1.1194x over previous
import jax
import jax.numpy as jnp
from jax import lax
from jax.experimental import pallas as pl
from jax.experimental.pallas import tpu as pltpu
from jax.experimental.pallas import tpu_sc as plsc

NN = 10000
NP = 10240
EE = 320000
FF = 16
NC = 2
NS = 16
NW = NC * NS
EPW = EE // NW
CH = 125
NCH = EPW // CH
RPT = NP // NS
NR = NP // 16
SPR = NR // NS
NID = NR // 128

_mesh = plsc.VectorSubcoreMesh(
    core_axis_name="c", subcore_axis_name="s", num_cores=NC, num_subcores=NS
)
_sc_params = pltpu.CompilerParams(
    use_tc_tiling_on_sc=False, needs_layout_passes=False
)
_sc_params_nl = _sc_params

KG = 4
NG = NCH // KG


def _rsqrt_nr(d):
    i = plsc.bitcast(d, jnp.int32)
    y = plsc.bitcast(
        jnp.full((16,), 0x5F3759DF, jnp.int32) - lax.shift_right_logical(i, 1),
        jnp.float32,
    )
    for _ in range(3):
        y = y * (1.5 - 0.5 * d * y * y)
    return y


def _dinv_packed(dp0_v, dp1_v, dpk_v):
    @pl.loop(0, SPR)
    def _(r):
        dpk_v[r, :] = _rsqrt_nr(1.0 + dp0_v[r] + dp1_v[r])


def _edge_pipeline(gsrc, src_v, dst_v, rows_v, acc_sh, gsem0, gsem1):
    def fire(g, half, sem):
        for b in range(KG):
            pltpu.async_copy(
                gsrc.at[src_v.at[g * KG + b]], rows_v.at[half * KG + b], sem
            )

    def drain(g, half, sem):
        for b in range(KG):
            pltpu.make_async_copy(
                gsrc.at[src_v.at[g * KG + b]], rows_v.at[half * KG + b], sem
            ).wait()

    def scat(g, half):
        for b in range(KG):
            pltpu.sync_copy(
                rows_v.at[half * KG + b], acc_sh.at[dst_v.at[g * KG + b]],
                add=True,
            )

    fire(0, 0, gsem0)

    @pl.loop(0, NG, step=2)
    def _pair(g):
        fire(g + 1, 1, gsem1)
        drain(g, 0, gsem0)
        scat(g, 0)

        @pl.when(g + 2 < NG)
        def _():
            fire(g + 2, 0, gsem0)

        drain(g + 1, 1, gsem1)
        scat(g + 1, 1)


def _mm1_body(x_ref, w1_ref, h1_ref):
    h1_ref[pl.ds(0, NN), :] = jnp.dot(
        x_ref[...], w1_ref[...], preferred_element_type=jnp.float32
    )
    h1_ref[pl.ds(NN, NP - NN), :] = jnp.zeros((NP - NN, FF), jnp.float32)


_mm1 = pl.pallas_call(
    _mm1_body, out_shape=jax.ShapeDtypeStruct((NP, FF), jnp.float32)
)


def _deg_body(edge_hbm, iden_hbm, zeros_hbm, out_hbm, dst_v, iden_v, loc_v,
              acc_sh, ssem):
    cid = lax.axis_index("c")
    sid = lax.axis_index("s")
    wid = sid * NC + cid
    pltpu.sync_copy(edge_hbm.at[1, wid], dst_v)
    pltpu.sync_copy(iden_hbm, iden_v)
    pltpu.sync_copy(zeros_hbm, acc_sh.at[pl.ds(sid * SPR, SPR)])

    @pl.loop(0, NR)
    def _zero(i):
        loc_v[i, :] = jnp.zeros((16,), jnp.float32)

    ones16 = jnp.ones((16,), jnp.float32)
    lanes = lax.iota(jnp.int32, 16)
    tail_mask = lanes < (CH - 112)

    @pl.loop(0, NCH)
    def _count(j):
        for c in range(0, 112, 16):
            dv = dst_v[j, pl.ds(c, 16)]
            plsc.addupdate_scatter(
                loc_v,
                [lax.shift_right_logical(dv, 4), lax.bitwise_and(dv, 15)],
                ones16,
            )
        dvt = plsc.load_gather(dst_v, [jnp.full((16,), j, jnp.int32),
                                       jnp.minimum(112 + lanes, CH - 1)])
        plsc.addupdate_scatter(
            loc_v,
            [lax.shift_right_logical(dvt, 4), lax.bitwise_and(dvt, 15)],
            ones16, mask=tail_mask,
        )

    plsc.subcore_barrier()

    for j in range(NID):
        pltpu.async_copy(
            loc_v.at[pl.ds(j * 128, 128)], acc_sh.at[iden_v.at[j]], ssem,
            add=True,
        )
    for j in range(NID):
        pltpu.make_async_copy(
            loc_v.at[pl.ds(j * 128, 128)], acc_sh.at[iden_v.at[j]], ssem
        ).wait()

    plsc.subcore_barrier()
    pltpu.sync_copy(
        acc_sh.at[pl.ds(sid * SPR, SPR)],
        out_hbm.at[cid, pl.ds(sid * SPR, SPR)],
    )


_deg_call = pl.kernel(
    _deg_body,
    out_type=jax.ShapeDtypeStruct((NC, NR, 16), jnp.float32),
    mesh=_mesh,
    scratch_types=[
        pltpu.VMEM((NCH, CH), jnp.int32),
        pltpu.VMEM((NID, 128), jnp.int32),
        pltpu.VMEM((NR, 16), jnp.float32),
        pltpu.VMEM_SHARED((NR, 16), jnp.float32),
        pltpu.SemaphoreType.DMA,
    ],
    compiler_params=_sc_params_nl,
)


def _agg1_body(edge_hbm, h1_hbm, degp_hbm, zeros_hbm, aggp_hbm, g1_hbm,
               src_v, dst_v, rows_v, dp0_v, dp1_v, dpk_v, g1_v, acc_sh, g_sh,
               gsem0, gsem1):
    cid = lax.axis_index("c")
    sid = lax.axis_index("s")
    wid = sid * NC + cid
    pltpu.sync_copy(edge_hbm.at[0, wid], src_v)
    pltpu.sync_copy(edge_hbm.at[1, wid], dst_v)
    pltpu.sync_copy(zeros_hbm, acc_sh.at[pl.ds(sid * RPT, RPT)])
    pltpu.sync_copy(degp_hbm.at[0, pl.ds(sid * SPR, SPR)], dp0_v)
    pltpu.sync_copy(degp_hbm.at[1, pl.ds(sid * SPR, SPR)], dp1_v)
    pltpu.sync_copy(h1_hbm.at[pl.ds(sid * RPT, RPT)], g1_v)
    _dinv_packed(dp0_v, dp1_v, dpk_v)

    @pl.loop(0, SPR)
    def _scale(r):
        drow = dpk_v[r, :]
        for c in range(16):
            n = r * 16 + c
            g1_v[n, :] = g1_v[n, :] * drow[c]

    pltpu.sync_copy(g1_v, g_sh.at[pl.ds(sid * RPT, RPT)])
    pltpu.sync_copy(g1_v, g1_hbm.at[pl.ds(sid * RPT, RPT)])
    plsc.subcore_barrier()

    _edge_pipeline(g_sh, src_v, dst_v, rows_v, acc_sh, gsem0, gsem1)

    plsc.subcore_barrier()
    pltpu.sync_copy(
        acc_sh.at[pl.ds(sid * RPT, RPT)],
        aggp_hbm.at[cid, pl.ds(sid * RPT, RPT)],
    )


_agg1_call = pl.kernel(
    _agg1_body,
    out_type=[
        jax.ShapeDtypeStruct((NC, NP, FF), jnp.float32),
        jax.ShapeDtypeStruct((NP, FF), jnp.float32),
    ],
    mesh=_mesh,
    scratch_types=[
        pltpu.VMEM((NCH, CH), jnp.int32),
        pltpu.VMEM((NCH, CH), jnp.int32),
        pltpu.VMEM((2 * KG, CH, FF), jnp.float32),
        pltpu.VMEM((SPR, 16), jnp.float32),
        pltpu.VMEM((SPR, 16), jnp.float32),
        pltpu.VMEM((SPR, 16), jnp.float32),
        pltpu.VMEM((RPT, FF), jnp.float32),
        pltpu.VMEM_SHARED((NP, FF), jnp.float32),
        pltpu.VMEM_SHARED((NP, FF), jnp.float32),
        pltpu.SemaphoreType.DMA,
        pltpu.SemaphoreType.DMA,
    ],
    compiler_params=_sc_params,
)


def _agg2_body(edge_hbm, degp_hbm, aggp1_hbm, g1_hbm, b1_hbm, w2_hbm,
               zeros_hbm, aggp_hbm, g2_hbm,
               src_v, dst_v, rows_v, dp0_v, dp1_v, dpk_v, p0_v, p1_v, g_v,
               b1_v, w2_v, acc_sh, g_sh, gsem0, gsem1):
    cid = lax.axis_index("c")
    sid = lax.axis_index("s")
    wid = sid * NC + cid
    pltpu.sync_copy(edge_hbm.at[0, wid], src_v)
    pltpu.sync_copy(edge_hbm.at[1, wid], dst_v)
    pltpu.sync_copy(zeros_hbm, acc_sh.at[pl.ds(sid * RPT, RPT)])
    pltpu.sync_copy(degp_hbm.at[0, pl.ds(sid * SPR, SPR)], dp0_v)
    pltpu.sync_copy(degp_hbm.at[1, pl.ds(sid * SPR, SPR)], dp1_v)
    pltpu.sync_copy(aggp1_hbm.at[0, pl.ds(sid * RPT, RPT)], p0_v)
    pltpu.sync_copy(aggp1_hbm.at[1, pl.ds(sid * RPT, RPT)], p1_v)
    pltpu.sync_copy(g1_hbm.at[pl.ds(sid * RPT, RPT)], g_v)
    pltpu.sync_copy(b1_hbm, b1_v)
    pltpu.sync_copy(w2_hbm, w2_v)
    _dinv_packed(dp0_v, dp1_v, dpk_v)

    b1vec = b1_v[...]
    w2rows = [w2_v[k, :] for k in range(FF)]

    @pl.loop(0, SPR)
    def _comb(r):
        drow = dpk_v[r, :]
        for c in range(16):
            n = r * 16 + c
            row = jnp.maximum(
                b1vec + drow[c] * (p0_v[n, :] + p1_v[n, :] + g_v[n, :]), 0.0
            )
            h2 = row[0] * w2rows[0]
            for k in range(1, FF):
                h2 = h2 + row[k] * w2rows[k]
            g_v[n, :] = h2 * drow[c]

    pltpu.sync_copy(g_v, g_sh.at[pl.ds(sid * RPT, RPT)])
    pltpu.sync_copy(g_v, g2_hbm.at[pl.ds(sid * RPT, RPT)])
    plsc.subcore_barrier()

    _edge_pipeline(g_sh, src_v, dst_v, rows_v, acc_sh, gsem0, gsem1)

    plsc.subcore_barrier()
    pltpu.sync_copy(
        acc_sh.at[pl.ds(sid * RPT, RPT)],
        aggp_hbm.at[cid, pl.ds(sid * RPT, RPT)],
    )


_agg2_call = pl.kernel(
    _agg2_body,
    out_type=[
        jax.ShapeDtypeStruct((NC, NP, FF), jnp.float32),
        jax.ShapeDtypeStruct((NP, FF), jnp.float32),
    ],
    mesh=_mesh,
    scratch_types=[
        pltpu.VMEM((NCH, CH), jnp.int32),
        pltpu.VMEM((NCH, CH), jnp.int32),
        pltpu.VMEM((2 * KG, CH, FF), jnp.float32),
        pltpu.VMEM((SPR, 16), jnp.float32),
        pltpu.VMEM((SPR, 16), jnp.float32),
        pltpu.VMEM((SPR, 16), jnp.float32),
        pltpu.VMEM((RPT, FF), jnp.float32),
        pltpu.VMEM((RPT, FF), jnp.float32),
        pltpu.VMEM((RPT, FF), jnp.float32),
        pltpu.VMEM((16,), jnp.float32),
        pltpu.VMEM((FF, FF), jnp.float32),
        pltpu.VMEM_SHARED((NP, FF), jnp.float32),
        pltpu.VMEM_SHARED((NP, FF), jnp.float32),
        pltpu.SemaphoreType.DMA,
        pltpu.SemaphoreType.DMA,
    ],
    compiler_params=_sc_params,
)


def _fin_body(degp_hbm, aggp2_hbm, g2_hbm, b2_hbm, fcv_hbm, out_hbm,
              dp0_v, dp1_v, dpk_v, p0_v, p1_v, g_v, b2_v, fcv_v,
              tot_v, pool_sh):
    cid = lax.axis_index("c")
    sid = lax.axis_index("s")
    pltpu.sync_copy(degp_hbm.at[0, pl.ds(sid * SPR, SPR)], dp0_v)
    pltpu.sync_copy(degp_hbm.at[1, pl.ds(sid * SPR, SPR)], dp1_v)
    pltpu.sync_copy(aggp2_hbm.at[0, pl.ds(sid * RPT, RPT)], p0_v)
    pltpu.sync_copy(aggp2_hbm.at[1, pl.ds(sid * RPT, RPT)], p1_v)
    pltpu.sync_copy(g2_hbm.at[pl.ds(sid * RPT, RPT)], g_v)
    pltpu.sync_copy(b2_hbm, b2_v)
    pltpu.sync_copy(fcv_hbm, fcv_v)
    _dinv_packed(dp0_v, dp1_v, dpk_v)

    lanes = lax.iota(jnp.int32, 16)
    mask8 = lanes < 8
    b2vec = jnp.where(mask8, plsc.load_gather(b2_v, [jnp.minimum(lanes, 7)]),
                      0.0)
    nrows = jnp.where(sid == NS - 1, (NN - (NS - 1) * RPT) // 16, SPR)

    def _body(r, acc):
        drow = dpk_v[r, :]
        for c in range(16):
            n = r * 16 + c
            row = jnp.maximum(
                b2vec + drow[c] * (p0_v[n, :] + p1_v[n, :] + g_v[n, :]), 0.0
            )
            acc = acc + row
        return acc

    pooled = lax.fori_loop(0, nrows, _body, jnp.zeros((16,), jnp.float32))
    tot_v[...] = pooled
    pltpu.sync_copy(tot_v, pool_sh.at[sid])
    plsc.subcore_barrier()

    @pl.when(sid == 0)
    def _():
        pltpu.sync_copy(pool_sh, g_v.at[pl.ds(0, NS)])
        total = jnp.zeros((16,), jnp.float32)
        for r in range(NS):
            total = total + g_v[r, :]
        total = total * (1.0 / NN)
        fcv = fcv_v[...]
        z = lax.reduce_sum_p.bind(
            total * jnp.where(mask8, fcv, 0.0), axes=(0,)
        ) + fcv[8]
        zv = jnp.zeros((16,), jnp.float32) + z
        sig = 1.0 / (1.0 + jnp.exp(-zv))
        tot_v[...] = sig
        pltpu.sync_copy(tot_v, out_hbm.at[cid])


_fin_call = pl.kernel(
    _fin_body,
    out_type=jax.ShapeDtypeStruct((NC, 16), jnp.float32),
    mesh=_mesh,
    scratch_types=[
        pltpu.VMEM((SPR, 16), jnp.float32),
        pltpu.VMEM((SPR, 16), jnp.float32),
        pltpu.VMEM((SPR, 16), jnp.float32),
        pltpu.VMEM((RPT, FF), jnp.float32),
        pltpu.VMEM((RPT, FF), jnp.float32),
        pltpu.VMEM((RPT, FF), jnp.float32),
        pltpu.VMEM((8,), jnp.float32),
        pltpu.VMEM((16,), jnp.float32),
        pltpu.VMEM((16,), jnp.float32),
        pltpu.VMEM_SHARED((NS, 16), jnp.float32),
    ],
    compiler_params=_sc_params,
)


@jax.jit
def kernel(x, edge_index, W1, b1, W2, b2, fcW, fcb):
    edge4 = edge_index.reshape(2, NW, NCH, CH)
    zeros_stripe = jnp.zeros((RPT, FF), jnp.float32)
    zeros_deg = jnp.zeros((SPR, 16), jnp.float32)
    iden = jnp.arange(NR, dtype=jnp.int32).reshape(NID, 128)

    degp = _deg_call(edge4, iden, zeros_deg)
    h1 = _mm1(x, W1)
    agg1p, g1 = _agg1_call(edge4, h1, degp, zeros_stripe)
    w2p = jnp.pad(W2, ((0, 0), (0, FF - W2.shape[1])))
    agg2p, g2 = _agg2_call(edge4, degp, agg1p, g1, b1, w2p, zeros_stripe)
    fcv = jnp.concatenate([fcW[:, 0], fcb, jnp.zeros((7,), jnp.float32)])
    sig = _fin_call(degp, agg2p, g2, b2, fcv)
    return sig[0, :1]
